# baseline (device time: 20164 ns/iter reference)
import jax
import jax.numpy as jnp
from jax import lax
from jax.experimental import pallas as pl
from jax.experimental.pallas import tpu as pltpu


def kernel(x):
    m, n = x.shape
    h = m // 2
    q = h // 2
    e = q // 2

    def body(x_ref, out_ref, comm_q, comm_e, send_sems, recv_sems):
        my_x = lax.axis_index("x")
        my_y = lax.axis_index("y")
        x_nbr = (1 - my_x, my_y)
        y_nbr = (my_x, 1 - my_y)

        barrier_sem = pltpu.get_barrier_semaphore()
        for nbr in (x_nbr, y_nbr):
            pl.semaphore_signal(
                barrier_sem, inc=1,
                device_id=nbr, device_id_type=pl.DeviceIdType.MESH,
            )
        pl.semaphore_wait(barrier_sem, 2)

        def xchg(src, dst, slot, nbr):
            return pltpu.make_async_remote_copy(
                src_ref=src,
                dst_ref=dst,
                send_sem=send_sems.at[slot],
                recv_sem=recv_sems.at[slot],
                device_id=nbr,
                device_id_type=pl.DeviceIdType.MESH,
            )

        keep0 = my_x * q
        base0 = keep0 + my_y * e
        keep1 = h + my_y * q
        base1 = keep1 + my_x * e

        rs1_0 = xchg(x_ref.at[pl.ds((1 - my_x) * q, q), :],
                     comm_q.at[0], 0, x_nbr)
        rs1_1 = xchg(x_ref.at[pl.ds(h + (1 - my_y) * q, q), :],
                     comm_q.at[1], 1, y_nbr)
        rs1_0.start()
        rs1_1.start()

        rs1_0.wait()
        out_ref[pl.ds(keep0, q), :] = (
            x_ref[pl.ds(keep0, q), :] + comm_q[0, :, :]
        )
        rs2_0 = xchg(out_ref.at[pl.ds(keep0 + (1 - my_y) * e, e), :],
                     comm_e.at[0], 2, y_nbr)
        rs2_0.start()

        rs1_1.wait()
        out_ref[pl.ds(keep1, q), :] = (
            x_ref[pl.ds(keep1, q), :] + comm_q[1, :, :]
        )
        rs2_1 = xchg(out_ref.at[pl.ds(keep1 + (1 - my_x) * e, e), :],
                     comm_e.at[1], 3, x_nbr)
        rs2_1.start()

        rs2_0.wait()
        out_ref[pl.ds(base0, e), :] = (
            out_ref[pl.ds(base0, e), :] + comm_e[0, :, :]
        )
        ag1_0 = xchg(out_ref.at[pl.ds(base0, e), :],
                     out_ref.at[pl.ds(base0, e), :], 4, y_nbr)
        ag1_0.start()

        rs2_1.wait()
        out_ref[pl.ds(base1, e), :] = (
            out_ref[pl.ds(base1, e), :] + comm_e[1, :, :]
        )
        ag1_1 = xchg(out_ref.at[pl.ds(base1, e), :],
                     out_ref.at[pl.ds(base1, e), :], 5, x_nbr)
        ag1_1.start()

        ag1_0.wait()
        ag2_0 = xchg(out_ref.at[pl.ds(keep0, q), :],
                     out_ref.at[pl.ds(keep0, q), :], 6, x_nbr)
        ag2_0.start()

        ag1_1.wait()
        ag2_1 = xchg(out_ref.at[pl.ds(keep1, q), :],
                     out_ref.at[pl.ds(keep1, q), :], 7, y_nbr)
        ag2_1.start()

        ag2_0.wait()
        ag2_1.wait()

    return pl.pallas_call(
        body,
        out_shape=jax.ShapeDtypeStruct((m, n), x.dtype),
        in_specs=[pl.BlockSpec(memory_space=pltpu.VMEM)],
        out_specs=pl.BlockSpec(memory_space=pltpu.VMEM),
        scratch_shapes=[
            pltpu.VMEM((2, q, n), x.dtype),
            pltpu.VMEM((2, e, n), x.dtype),
            pltpu.SemaphoreType.DMA((8,)),
            pltpu.SemaphoreType.DMA((8,)),
        ],
        compiler_params=pltpu.CompilerParams(collective_id=0),
    )(x)


# device time: 18646 ns/iter; 1.0814x vs baseline; 1.0814x over previous
import jax
import jax.numpy as jnp
from jax import lax
from jax.experimental import pallas as pl
from jax.experimental.pallas import tpu as pltpu

CHUNKS = 4


def kernel(x):
    m, n = x.shape
    c = m // (2 * CHUNKS)
    n_slots = 4 * CHUNKS

    def body(x_ref, out_ref, comm_ref, send_sems, recv_sems):
        my_x = lax.axis_index("x")
        my_y = lax.axis_index("y")
        x_nbr = (1 - my_x, my_y)
        y_nbr = (my_x, 1 - my_y)

        barrier_sem = pltpu.get_barrier_semaphore()
        for nbr in (x_nbr, y_nbr):
            pl.semaphore_signal(
                barrier_sem, inc=1,
                device_id=nbr, device_id_type=pl.DeviceIdType.MESH,
            )
        pl.semaphore_wait(barrier_sem, 2)

        def exchange(src, slot, nbr):
            return pltpu.make_async_remote_copy(
                src_ref=src,
                dst_ref=comm_ref.at[slot],
                send_sem=send_sems.at[slot],
                recv_sem=recv_sems.at[slot],
                device_id=nbr,
                device_id_type=pl.DeviceIdType.MESH,
            )

        p1 = []
        for k in range(CHUNKS):
            p1.append((exchange(x_ref.at[pl.ds(k * c, c), :], k, x_nbr),
                       k * c, k))
            p1.append((exchange(
                x_ref.at[pl.ds(m // 2 + k * c, c), :], CHUNKS + k, y_nbr),
                m // 2 + k * c, CHUNKS + k))
        for rdma, _, _ in p1:
            rdma.start()

        p2 = []
        for rdma, row, slot in p1:
            rdma.wait()
            out_ref[pl.ds(row, c), :] = (
                x_ref[pl.ds(row, c), :] + comm_ref[slot, :, :]
            )
            nbr2 = y_nbr if slot < CHUNKS else x_nbr
            rdma2 = exchange(out_ref.at[pl.ds(row, c), :], 2 * CHUNKS + slot,
                             nbr2)
            rdma2.start()
            p2.append((rdma2, row, 2 * CHUNKS + slot))

        for rdma2, row, slot in p2:
            rdma2.wait()
            out_ref[pl.ds(row, c), :] = (
                out_ref[pl.ds(row, c), :] + comm_ref[slot, :, :]
            )

    return pl.pallas_call(
        body,
        out_shape=jax.ShapeDtypeStruct((m, n), x.dtype),
        in_specs=[pl.BlockSpec(memory_space=pltpu.VMEM)],
        out_specs=pl.BlockSpec(memory_space=pltpu.VMEM),
        scratch_shapes=[
            pltpu.VMEM((n_slots, c, n), x.dtype),
            pltpu.SemaphoreType.DMA((n_slots,)),
            pltpu.SemaphoreType.DMA((n_slots,)),
        ],
        compiler_params=pltpu.CompilerParams(collective_id=0),
    )(x)


# device time: 18513 ns/iter; 1.0892x vs baseline; 1.0072x over previous
import jax
import jax.numpy as jnp
from jax import lax
from jax.experimental import pallas as pl
from jax.experimental.pallas import tpu as pltpu

CHUNKS = 2


def kernel(x):
    m, n = x.shape
    c = m // (2 * CHUNKS)
    n_slots = 4 * CHUNKS

    def body(x_ref, out_ref, comm_ref, send_sems, recv_sems):
        my_x = lax.axis_index("x")
        my_y = lax.axis_index("y")
        x_nbr = (1 - my_x, my_y)
        y_nbr = (my_x, 1 - my_y)

        barrier_sem = pltpu.get_barrier_semaphore()
        for nbr in (x_nbr, y_nbr):
            pl.semaphore_signal(
                barrier_sem, inc=1,
                device_id=nbr, device_id_type=pl.DeviceIdType.MESH,
            )
        pl.semaphore_wait(barrier_sem, 2)

        def exchange(src, slot, nbr):
            return pltpu.make_async_remote_copy(
                src_ref=src,
                dst_ref=comm_ref.at[slot],
                send_sem=send_sems.at[slot],
                recv_sem=recv_sems.at[slot],
                device_id=nbr,
                device_id_type=pl.DeviceIdType.MESH,
            )

        p1 = []
        for k in range(CHUNKS):
            p1.append((exchange(x_ref.at[pl.ds(k * c, c), :], k, x_nbr),
                       k * c, k))
            p1.append((exchange(
                x_ref.at[pl.ds(m // 2 + k * c, c), :], CHUNKS + k, y_nbr),
                m // 2 + k * c, CHUNKS + k))
        for rdma, _, _ in p1:
            rdma.start()

        p2 = []
        for rdma, row, slot in p1:
            rdma.wait_recv()
            out_ref[pl.ds(row, c), :] = (
                x_ref[pl.ds(row, c), :] + comm_ref[slot, :, :]
            )
            nbr2 = y_nbr if slot < CHUNKS else x_nbr
            rdma2 = exchange(out_ref.at[pl.ds(row, c), :], 2 * CHUNKS + slot,
                             nbr2)
            rdma2.start()
            p2.append((rdma2, row, 2 * CHUNKS + slot))

        for rdma2, row, slot in p2:
            rdma2.wait()
            out_ref[pl.ds(row, c), :] = (
                out_ref[pl.ds(row, c), :] + comm_ref[slot, :, :]
            )

        for rdma, _, _ in p1:
            rdma.wait_send()

    return pl.pallas_call(
        body,
        out_shape=jax.ShapeDtypeStruct((m, n), x.dtype),
        in_specs=[pl.BlockSpec(memory_space=pltpu.VMEM)],
        out_specs=pl.BlockSpec(memory_space=pltpu.VMEM),
        scratch_shapes=[
            pltpu.VMEM((n_slots, c, n), x.dtype),
            pltpu.SemaphoreType.DMA((n_slots,)),
            pltpu.SemaphoreType.DMA((n_slots,)),
        ],
        compiler_params=pltpu.CompilerParams(collective_id=0),
    )(x)
